# baseline (speedup 1.0000x reference)
import functools

import jax
import jax.numpy as jnp
from jax import lax
from jax.experimental import pallas as pl
from jax.experimental.pallas import tpu as pltpu
from jax.experimental.pallas import tpu_sc as plsc

N = 50000
E = 800000
V = 100000
G = 512
D = 64
DH = 128

NC = 2
NS = 16
L = 16
NW = NC * NS

HALF = N // NC
HP = 25088
PN = 2 * HP
DUMP = HALF
SLAB = PN // NW

CE = 128
CB = 2048
EPT = 51200
EP = NS * EPT
NB = EPT // CB
SCH = CB // CE
TB = EP // CB

OUTR = 520


def _mesh():
    return plsc.VectorSubcoreMesh(core_axis_name="c", subcore_axis_name="s",
                                  num_cores=NC, num_subcores=NS)


_SC_PARAMS = pltpu.CompilerParams(use_tc_tiling_on_sc=False,
                                  needs_layout_passes=False)


def _bcast(vec, j):
    idx = jnp.full((L, 1), j, jnp.int32)
    dnums = lax.GatherDimensionNumbers(
        offset_dims=(), collapsed_slice_dims=(0,), start_index_map=(0,))
    return lax.gather(vec, idx, dnums, (1,),
                      mode=lax.GatherScatterMode.PROMISE_IN_BOUNDS)


def _embed_gather(embed, x_pad):
    GC = 112
    NG = SLAB // GC

    @functools.partial(
        pl.kernel,
        out_type=jax.ShapeDtypeStruct((PN, D), jnp.float32),
        mesh=_mesh(),
        compiler_params=_SC_PARAMS,
        scratch_types=[
            pltpu.VMEM((GC,), jnp.int32),
            pltpu.VMEM((GC, D), jnp.float32),
            pltpu.SemaphoreType.DMA,
        ],
    )
    def k(embed_hbm, x_hbm, h_hbm, idx_v, rows_v, sem):
        c = lax.axis_index("c")
        s = lax.axis_index("s")
        base = (s * NC + c) * SLAB

        def body(g, carry):
            off = base + g * GC
            pltpu.sync_copy(x_hbm.at[pl.ds(off, GC)], idx_v)
            pltpu.async_copy(embed_hbm.at[idx_v], rows_v, sem).wait()
            pltpu.sync_copy(rows_v, h_hbm.at[pl.ds(off, GC)])
            return carry

        lax.fori_loop(0, NG, body, 0)

    return k(embed, x_pad)


def _edge_agg(table, epack, dep):
    EC = 64
    ECH = CB // EC
    ZC = 112
    NZ = (HP // NS) // ZC

    @functools.partial(
        pl.kernel,
        out_type=jax.ShapeDtypeStruct((PN, D), jnp.float32),
        mesh=_mesh(),
        compiler_params=_SC_PARAMS,
        scratch_types=[
            pltpu.VMEM((1, 3, CB), jnp.int32),
            pltpu.VMEM((EC, D), jnp.float32),
            pltpu.VMEM((EC, D), jnp.float32),
            pltpu.VMEM((EC, D), jnp.float32),
            pltpu.VMEM((EC, D), jnp.float32),
            pltpu.VMEM((EC,), jnp.int32),
            pltpu.VMEM((EC,), jnp.int32),
            pltpu.VMEM_SHARED((HP, D), jnp.float32),
            pltpu.SemaphoreType.DMA,
            pltpu.SemaphoreType.DMA,
            pltpu.SemaphoreType.DMA,
            pltpu.SemaphoreType.DMA,
        ],
    )
    def k(tab_hbm, ep_hbm, dep_hbm, out_hbm, ebuf, rows0, rows1, msg0, msg1,
          idx0, idx1, agg, gsem0, gsem1, ssem0, ssem1):
        del dep_hbm
        c = lax.axis_index("c")
        s = lax.axis_index("s")
        base_c = c * HALF
        rows_ = (rows0, rows1)
        msgs = (msg0, msg1)
        idxs = (idx0, idx1)
        gsems = (gsem0, gsem1)
        ssems = (ssem0, ssem1)
        zero = jnp.zeros((L,), jnp.float32)

        def zrow(r, carry):
            for kk in range(D // L):
                msg0[r, pl.ds(kk * L, L)] = zero
            return carry

        lax.fori_loop(0, EC, zrow, 0)

        def zcopy(z, carry):
            pltpu.sync_copy(msg0.at[pl.ds(0, EC)],
                            agg.at[pl.ds(s * (HP // NS) + z * EC, EC)])
            return carry

        lax.fori_loop(0, (HP // NS) // EC, zcopy, 0)

        plsc.subcore_barrier()

        def gstart(par, sub):
            off = pl.multiple_of(sub * EC, EC)
            pltpu.async_copy(tab_hbm.at[ebuf.at[0, 0, pl.ds(off, EC)]],
                             rows_[par], gsems[par])

        def gwait(par):
            pltpu.make_async_copy(tab_hbm.at[idxs[par]], rows_[par],
                                  gsems[par]).wait()

        def sstart(par):
            pltpu.async_copy(msgs[par], agg.at[idxs[par]], ssems[par],
                             add=True)

        def swait(par):
            pltpu.make_async_copy(msgs[par], agg.at[idxs[par]],
                                  ssems[par]).wait()

        def compute(par, sub):
            for i in range(EC // L):
                sl = pl.ds(sub * EC + i * L, L)
                d16 = ebuf[0, 1, sl]
                inr = (d16 >= base_c) & (d16 < base_c + HALF)
                idxs[par][pl.ds(i * L, L)] = jnp.where(inr, d16 - base_c,
                                                       DUMP)
                w16 = plsc.bitcast(ebuf[0, 2, sl], jnp.float32)
                for j in range(L):
                    wj = _bcast(w16, j)
                    r = i * L + j
                    for kk in range(D // L):
                        msgs[par][r, pl.ds(kk * L, L)] = (
                            rows_[par][r, pl.ds(kk * L, L)] * wj)

        def blk(b, carry):
            blkid = s * NB + b
            pltpu.sync_copy(ep_hbm.at[pl.ds(blkid, 1)], ebuf)
            gstart(0, 0)
            gstart(1, 1)

            def pair(q, carry2):
                sub_a = q * 2
                sub_b = sub_a + 1

                @pl.when((b + q) > 0)
                def _():
                    swait(0)

                gwait(0)
                compute(0, sub_a)

                @pl.when(q < (ECH // 2 - 1))
                def _():
                    gstart(0, sub_a + 2)

                sstart(0)

                @pl.when((b + q) > 0)
                def _():
                    swait(1)

                gwait(1)
                compute(1, sub_b)

                @pl.when(q < (ECH // 2 - 1))
                def _():
                    gstart(1, sub_b + 2)

                sstart(1)
                return carry2

            lax.fori_loop(0, ECH // 2, pair, 0)
            return carry

        lax.fori_loop(0, NB, blk, 0)
        swait(0)
        swait(1)

        plsc.subcore_barrier()
        rows_pt = HP // NS
        pltpu.sync_copy(agg.at[pl.ds(s * rows_pt, rows_pt)],
                        out_hbm.at[pl.ds(c * HP + s * rows_pt, rows_pt)])

    return k(table, epack, dep)


def _deg(epack):
    ZC = 112
    NZ = (HP // NS) // ZC
    NP = 4

    @functools.partial(
        pl.kernel,
        out_type=jax.ShapeDtypeStruct((PN, L), jnp.float32),
        mesh=_mesh(),
        compiler_params=_SC_PARAMS,
        scratch_types=[
            pltpu.VMEM((1, 1, CB), jnp.int32),
            pltpu.VMEM((CE, L), jnp.float32),
            pltpu.VMEM((CE,), jnp.int32),
            pltpu.VMEM((CE,), jnp.int32),
            pltpu.VMEM((CE,), jnp.int32),
            pltpu.VMEM((CE,), jnp.int32),
            pltpu.VMEM_SHARED((HP, L), jnp.float32),
            pltpu.SemaphoreType.DMA,
            pltpu.SemaphoreType.DMA,
            pltpu.SemaphoreType.DMA,
            pltpu.SemaphoreType.DMA,
        ],
    )
    def k(ep_hbm, out_hbm, dbuf, onesb, idx0, idx1, idx2, idx3, degT,
          ssem0, ssem1, ssem2, ssem3):
        c = lax.axis_index("c")
        s = lax.axis_index("s")
        base_c = c * HALF
        idxs = (idx0, idx1, idx2, idx3)
        ssems = (ssem0, ssem1, ssem2, ssem3)
        zero = jnp.zeros((L,), jnp.float32)

        def zrow(r, carry):
            onesb[r, pl.ds(0, L)] = zero
            return carry

        lax.fori_loop(0, CE, zrow, 0)

        def zcopy(z, carry):
            pltpu.sync_copy(onesb.at[pl.ds(0, ZC)],
                            degT.at[pl.ds(s * (HP // NS) + z * ZC, ZC)])
            return carry

        lax.fori_loop(0, NZ, zcopy, 0)

        ones = jnp.ones((L,), jnp.float32)

        def orow(r, carry):
            onesb[r, pl.ds(0, L)] = ones
            return carry

        lax.fori_loop(0, CE, orow, 0)

        plsc.subcore_barrier()

        def sstart(par):
            pltpu.async_copy(onesb, degT.at[idxs[par]], ssems[par], add=True)

        def swait(par):
            pltpu.make_async_copy(onesb, degT.at[idxs[par]],
                                  ssems[par]).wait()

        def compute(par, sub):
            for i in range(CE // L):
                d16 = dbuf[0, 0, pl.ds(sub * CE + i * L, L)]
                inr = (d16 >= base_c) & (d16 < base_c + HALF)
                idxs[par][pl.ds(i * L, L)] = jnp.where(inr, d16 - base_c,
                                                       DUMP)

        def blk(b, carry):
            blkid = s * NB + b
            pltpu.sync_copy(ep_hbm.at[pl.ds(blkid, 1), pl.ds(1, 1)], dbuf)

            def quad(q, carry2):
                for p in range(NP):
                    @pl.when((b + q) > 0)
                    def _():
                        swait(p)

                    compute(p, q * NP + p)
                    sstart(p)
                return carry2

            lax.fori_loop(0, SCH // NP, quad, 0)
            return carry

        lax.fori_loop(0, NB, blk, 0)
        for p in range(NP):
            swait(p)

        plsc.subcore_barrier()
        rows_pt = HP // NS
        pltpu.sync_copy(degT.at[pl.ds(s * rows_pt, rows_pt)],
                        out_hbm.at[pl.ds(c * HP + s * rows_pt, rows_pt)])

    return k(epack)


def _dense_mid(agg1, degr, h, W1l, b1, W1r, W2l, b2, W2r):
    BR = 128
    GR = PN // BR

    def body(a1_ref, dg_ref, h_ref, w1l_ref, b1_ref, w1r_ref, w2l_ref,
             b2_ref, w2r_ref, p_ref, qb_ref):
        a1 = a1_ref[...]
        deg = jnp.maximum(dg_ref[...][:, :1], 1.0)
        mean = a1 / deg
        h1 = jnp.dot(mean, w1l_ref[...], preferred_element_type=jnp.float32)
        h1 = h1 + b1_ref[...]
        h1 = h1 + jnp.dot(h_ref[...], w1r_ref[...],
                          preferred_element_type=jnp.float32)
        h1 = jnp.maximum(h1, 0.0)
        p_ref[...] = jnp.dot(h1, w2l_ref[...],
                             preferred_element_type=jnp.float32)
        qb_ref[...] = jnp.dot(h1, w2r_ref[...],
                              preferred_element_type=jnp.float32) + b2_ref[...]

    return pl.pallas_call(
        body,
        grid=(GR,),
        in_specs=[
            pl.BlockSpec((BR, D), lambda i: (i, 0)),
            pl.BlockSpec((BR, L), lambda i: (i, 0)),
            pl.BlockSpec((BR, D), lambda i: (i, 0)),
            pl.BlockSpec((D, DH), lambda i: (0, 0)),
            pl.BlockSpec((1, DH), lambda i: (0, 0)),
            pl.BlockSpec((D, DH), lambda i: (0, 0)),
            pl.BlockSpec((DH, D), lambda i: (0, 0)),
            pl.BlockSpec((1, D), lambda i: (0, 0)),
            pl.BlockSpec((DH, D), lambda i: (0, 0)),
        ],
        out_specs=[pl.BlockSpec((BR, D), lambda i: (i, 0)),
                   pl.BlockSpec((BR, D), lambda i: (i, 0))],
        out_shape=[jax.ShapeDtypeStruct((PN, D), jnp.float32),
                   jax.ShapeDtypeStruct((PN, D), jnp.float32)],
    )(agg1, degr, h, W1l, b1.reshape(1, DH), W1r, W2l, b2.reshape(1, D),
      W2r)


def _pool(degr, agg2, qb, batch_pad):
    NSL = SLAB // L
    NPR = NSL // 2

    @functools.partial(
        pl.kernel,
        out_type=jax.ShapeDtypeStruct((NW, OUTR, D), jnp.float32),
        mesh=_mesh(),
        compiler_params=_SC_PARAMS,
        scratch_types=[
            pltpu.VMEM((L, D), jnp.float32),
            pltpu.VMEM((L, D), jnp.float32),
            pltpu.VMEM((L, D), jnp.float32),
            pltpu.VMEM((L, D), jnp.float32),
            pltpu.VMEM((L, L), jnp.float32),
            pltpu.VMEM((L, L), jnp.float32),
            pltpu.VMEM((L,), jnp.int32),
            pltpu.VMEM((L,), jnp.int32),
            pltpu.VMEM((OUTR, D), jnp.float32),
            pltpu.SemaphoreType.DMA,
            pltpu.SemaphoreType.DMA,
        ],
    )
    def k(dg_hbm, a2_hbm, qb_hbm, b_hbm, out_hbm, a2b0, a2b1, qb0, qb1,
          dg0, dg1, bb0, bb1, outl, psem0, psem1):
        c = lax.axis_index("c")
        s = lax.axis_index("s")
        wid = s * NC + c
        base = wid * SLAB
        a2bs = (a2b0, a2b1)
        qbs = (qb0, qb1)
        dgs = (dg0, dg1)
        bbs = (bb0, bb1)
        psems = (psem0, psem1)
        lanes = lax.iota(jnp.int32, L)
        ninf = jnp.full((L,), -jnp.inf, jnp.float32)

        def irow(r, carry):
            for kk in range(D // L):
                outl[r, pl.ds(kk * L, L)] = ninf
            return carry

        lax.fori_loop(0, OUTR, irow, 0)

        def start4(par, slab):
            row0 = base + slab * L
            pltpu.async_copy(a2_hbm.at[pl.ds(row0, L)], a2bs[par],
                             psems[par])
            pltpu.async_copy(qb_hbm.at[pl.ds(row0, L)], qbs[par],
                             psems[par])
            pltpu.async_copy(dg_hbm.at[pl.ds(row0, L)], dgs[par],
                             psems[par])
            pltpu.async_copy(b_hbm.at[pl.ds(row0, L)], bbs[par], psems[par])

        def wait4(par):
            pltpu.make_async_copy(a2_hbm.at[pl.ds(0, L)], a2bs[par],
                                  psems[par]).wait()
            pltpu.make_async_copy(qb_hbm.at[pl.ds(0, L)], qbs[par],
                                  psems[par]).wait()
            pltpu.make_async_copy(dg_hbm.at[pl.ds(0, L)], dgs[par],
                                  psems[par]).wait()
            pltpu.make_async_copy(b_hbm.at[pl.ds(0, L)], bbs[par],
                                  psems[par]).wait()

        def compute(par):
            deg16 = plsc.load_gather(dgs[par],
                                     [lanes, jnp.full((L,), 0, jnp.int32)])
            dinv16 = 1.0 / jnp.maximum(deg16, 1.0)
            b16 = bbs[par][...]
            for j in range(L):
                gj = _bcast(b16, j)
                dj = _bcast(dinv16, j)
                for kk in range(D // L):
                    cols = lanes + kk * L
                    h2v = (a2bs[par][j, pl.ds(kk * L, L)] * dj
                           + qbs[par][j, pl.ds(kk * L, L)])
                    cur = plsc.load_gather(outl, [gj, cols])
                    plsc.store_scatter(outl, [gj, cols],
                                       jnp.maximum(cur, h2v))

        start4(0, 0)

        def pairloop(i, carry):
            wait4(0)
            start4(1, 2 * i + 1)
            compute(0)
            wait4(1)

            @pl.when(i < (NPR - 1))
            def _():
                start4(0, 2 * i + 2)

            compute(1)
            return carry

        lax.fori_loop(0, NPR, pairloop, 0)
        pltpu.sync_copy(outl, out_hbm.at[wid])

    return k(degr, agg2, qb, batch_pad)


def _final_max(parts):
    def body(p_ref, o_ref):
        i = pl.program_id(0)

        @pl.when(i == 0)
        def _():
            o_ref[...] = jnp.full((G, D), -jnp.inf, jnp.float32)

        o_ref[...] = jnp.maximum(o_ref[...], p_ref[0, :G, :])

    return pl.pallas_call(
        body,
        grid=(NW,),
        in_specs=[pl.BlockSpec((1, OUTR, D), lambda i: (i, 0, 0))],
        out_specs=pl.BlockSpec((G, D), lambda i: (0, 0)),
        out_shape=jax.ShapeDtypeStruct((G, D), jnp.float32),
    )(parts)


def kernel(x, edge_index, edge_attr, batch, embed, W1l, b1, W1r, W2l, b2, W2r):
    x = x.astype(jnp.int32)
    src = edge_index[0].astype(jnp.int32)
    dst = edge_index[1].astype(jnp.int32)
    batch = batch.astype(jnp.int32)

    pad_shift = HP - HALF
    x_pad = jnp.zeros((PN,), jnp.int32)
    x_pad = x_pad.at[0:HALF].set(x[:HALF]).at[HP:HP + HALF].set(x[HALF:])
    batch_pad = jnp.full((PN,), G, jnp.int32)
    batch_pad = batch_pad.at[0:HALF].set(batch[:HALF])
    batch_pad = batch_pad.at[HP:HP + HALF].set(batch[HALF:])

    psrc = src + pad_shift * (src >= HALF).astype(jnp.int32)
    npad = EP - E
    psrc = jnp.concatenate([psrc, jnp.zeros((npad,), jnp.int32)])
    dstp = jnp.concatenate([dst, jnp.full((npad,), N, jnp.int32)])
    wp = jnp.concatenate([edge_attr, jnp.zeros((npad,), jnp.float32)])
    epack = jnp.stack(
        [psrc.reshape(TB, CB), dstp.reshape(TB, CB),
         lax.bitcast_convert_type(wp, jnp.int32).reshape(TB, CB)], axis=1)

    h = _embed_gather(embed, x_pad)
    degr = _deg(epack)
    agg1 = _edge_agg(h, epack, degr)
    p, qb = _dense_mid(agg1, degr, h, W1l, b1, W1r, W2l, b2, W2r)
    agg2 = _edge_agg(p, epack, degr)
    parts = _pool(degr, agg2, qb, batch_pad)
    return _final_max(parts)


# speedup vs baseline: 1.0002x; 1.0002x over previous
import functools

import jax
import jax.numpy as jnp
from jax import lax
from jax.experimental import pallas as pl
from jax.experimental.pallas import tpu as pltpu
from jax.experimental.pallas import tpu_sc as plsc

N = 50000
E = 800000
V = 100000
G = 512
D = 64
DH = 128

NC = 2
NS = 16
L = 16
NW = NC * NS

HALF = N // NC
HP = 25088
PN = 2 * HP
DUMP = HALF
SLAB = PN // NW

CE = 128
CB = 2048
EPT = 51200
EP = NS * EPT
NB = EPT // CB
SCH = CB // CE
TB = EP // CB

OUTR = 520


def _mesh():
    return plsc.VectorSubcoreMesh(core_axis_name="c", subcore_axis_name="s",
                                  num_cores=NC, num_subcores=NS)


_SC_PARAMS = pltpu.CompilerParams(use_tc_tiling_on_sc=False,
                                  needs_layout_passes=False)


def _bcast(vec, j):
    idx = jnp.full((L, 1), j, jnp.int32)
    dnums = lax.GatherDimensionNumbers(
        offset_dims=(), collapsed_slice_dims=(0,), start_index_map=(0,))
    return lax.gather(vec, idx, dnums, (1,),
                      mode=lax.GatherScatterMode.PROMISE_IN_BOUNDS)


def _embed_gather(embed, x_pad):
    GC = 112
    NG = SLAB // GC

    @functools.partial(
        pl.kernel,
        out_type=jax.ShapeDtypeStruct((PN, D), jnp.float32),
        mesh=_mesh(),
        compiler_params=_SC_PARAMS,
        scratch_types=[
            pltpu.VMEM((GC,), jnp.int32),
            pltpu.VMEM((GC, D), jnp.float32),
            pltpu.SemaphoreType.DMA,
        ],
    )
    def k(embed_hbm, x_hbm, h_hbm, idx_v, rows_v, sem):
        c = lax.axis_index("c")
        s = lax.axis_index("s")
        base = (s * NC + c) * SLAB

        def body(g, carry):
            off = base + g * GC
            pltpu.sync_copy(x_hbm.at[pl.ds(off, GC)], idx_v)
            pltpu.async_copy(embed_hbm.at[idx_v], rows_v, sem).wait()
            pltpu.sync_copy(rows_v, h_hbm.at[pl.ds(off, GC)])
            return carry

        lax.fori_loop(0, NG, body, 0)

    return k(embed, x_pad)


def _edge_agg(table, epack, dep):
    EC = 64
    ECH = CB // EC
    ZC = 112
    NZ = (HP // NS) // ZC

    @functools.partial(
        pl.kernel,
        out_type=jax.ShapeDtypeStruct((PN, D), jnp.float32),
        mesh=_mesh(),
        compiler_params=_SC_PARAMS,
        scratch_types=[
            pltpu.VMEM((1, 3, CB), jnp.int32),
            pltpu.VMEM((EC, D), jnp.float32),
            pltpu.VMEM((EC, D), jnp.float32),
            pltpu.VMEM((EC, D), jnp.float32),
            pltpu.VMEM((EC, D), jnp.float32),
            pltpu.VMEM((EC,), jnp.int32),
            pltpu.VMEM((EC,), jnp.int32),
            pltpu.VMEM_SHARED((HP, D), jnp.float32),
            pltpu.SemaphoreType.DMA,
            pltpu.SemaphoreType.DMA,
            pltpu.SemaphoreType.DMA,
            pltpu.SemaphoreType.DMA,
        ],
    )
    def k(tab_hbm, ep_hbm, dep_hbm, out_hbm, ebuf, rows0, rows1, msg0, msg1,
          idx0, idx1, agg, gsem0, gsem1, ssem0, ssem1):
        del dep_hbm
        c = lax.axis_index("c")
        s = lax.axis_index("s")
        base_c = c * HALF
        rows_ = (rows0, rows1)
        msgs = (msg0, msg1)
        idxs = (idx0, idx1)
        gsems = (gsem0, gsem1)
        ssems = (ssem0, ssem1)
        zero = jnp.zeros((L,), jnp.float32)

        def zrow(r, carry):
            for kk in range(D // L):
                msg0[r, pl.ds(kk * L, L)] = zero
            return carry

        lax.fori_loop(0, EC, zrow, 0)

        ZCE = 56

        def zcopy(z, carry):
            pltpu.sync_copy(msg0.at[pl.ds(0, ZCE)],
                            agg.at[pl.ds(s * (HP // NS) + z * ZCE, ZCE)])
            return carry

        lax.fori_loop(0, (HP // NS) // ZCE, zcopy, 0)

        plsc.subcore_barrier()

        def gstart(par, sub):
            off = pl.multiple_of(sub * EC, EC)
            pltpu.async_copy(tab_hbm.at[ebuf.at[0, 0, pl.ds(off, EC)]],
                             rows_[par], gsems[par])

        def gwait(par):
            pltpu.make_async_copy(tab_hbm.at[idxs[par]], rows_[par],
                                  gsems[par]).wait()

        def sstart(par):
            pltpu.async_copy(msgs[par], agg.at[idxs[par]], ssems[par],
                             add=True)

        def swait(par):
            pltpu.make_async_copy(msgs[par], agg.at[idxs[par]],
                                  ssems[par]).wait()

        def compute(par, sub):
            for i in range(EC // L):
                sl = pl.ds(sub * EC + i * L, L)
                d16 = ebuf[0, 1, sl]
                inr = (d16 >= base_c) & (d16 < base_c + HALF)
                idxs[par][pl.ds(i * L, L)] = jnp.where(inr, d16 - base_c,
                                                       DUMP)
                w16 = plsc.bitcast(ebuf[0, 2, sl], jnp.float32)
                for j in range(L):
                    wj = _bcast(w16, j)
                    r = i * L + j
                    for kk in range(D // L):
                        msgs[par][r, pl.ds(kk * L, L)] = (
                            rows_[par][r, pl.ds(kk * L, L)] * wj)

        def blk(b, carry):
            blkid = s * NB + b
            pltpu.sync_copy(ep_hbm.at[pl.ds(blkid, 1)], ebuf)
            gstart(0, 0)
            gstart(1, 1)

            def pair(q, carry2):
                sub_a = q * 2
                sub_b = sub_a + 1

                @pl.when((b + q) > 0)
                def _():
                    swait(0)

                gwait(0)
                compute(0, sub_a)

                @pl.when(q < (ECH // 2 - 1))
                def _():
                    gstart(0, sub_a + 2)

                sstart(0)

                @pl.when((b + q) > 0)
                def _():
                    swait(1)

                gwait(1)
                compute(1, sub_b)

                @pl.when(q < (ECH // 2 - 1))
                def _():
                    gstart(1, sub_b + 2)

                sstart(1)
                return carry2

            lax.fori_loop(0, ECH // 2, pair, 0)
            return carry

        lax.fori_loop(0, NB, blk, 0)
        swait(0)
        swait(1)

        plsc.subcore_barrier()
        rows_pt = HP // NS
        pltpu.sync_copy(agg.at[pl.ds(s * rows_pt, rows_pt)],
                        out_hbm.at[pl.ds(c * HP + s * rows_pt, rows_pt)])

    return k(table, epack, dep)


def _deg(epack):
    ZC = 112
    NZ = (HP // NS) // ZC
    NP = 4

    @functools.partial(
        pl.kernel,
        out_type=jax.ShapeDtypeStruct((PN, L), jnp.float32),
        mesh=_mesh(),
        compiler_params=_SC_PARAMS,
        scratch_types=[
            pltpu.VMEM((1, 1, CB), jnp.int32),
            pltpu.VMEM((CE, L), jnp.float32),
            pltpu.VMEM((CE,), jnp.int32),
            pltpu.VMEM((CE,), jnp.int32),
            pltpu.VMEM((CE,), jnp.int32),
            pltpu.VMEM((CE,), jnp.int32),
            pltpu.VMEM_SHARED((HP, L), jnp.float32),
            pltpu.SemaphoreType.DMA,
            pltpu.SemaphoreType.DMA,
            pltpu.SemaphoreType.DMA,
            pltpu.SemaphoreType.DMA,
        ],
    )
    def k(ep_hbm, out_hbm, dbuf, onesb, idx0, idx1, idx2, idx3, degT,
          ssem0, ssem1, ssem2, ssem3):
        c = lax.axis_index("c")
        s = lax.axis_index("s")
        base_c = c * HALF
        idxs = (idx0, idx1, idx2, idx3)
        ssems = (ssem0, ssem1, ssem2, ssem3)
        zero = jnp.zeros((L,), jnp.float32)

        def zrow(r, carry):
            onesb[r, pl.ds(0, L)] = zero
            return carry

        lax.fori_loop(0, CE, zrow, 0)

        def zcopy(z, carry):
            pltpu.sync_copy(onesb.at[pl.ds(0, ZC)],
                            degT.at[pl.ds(s * (HP // NS) + z * ZC, ZC)])
            return carry

        lax.fori_loop(0, NZ, zcopy, 0)

        ones = jnp.ones((L,), jnp.float32)

        def orow(r, carry):
            onesb[r, pl.ds(0, L)] = ones
            return carry

        lax.fori_loop(0, CE, orow, 0)

        plsc.subcore_barrier()

        def sstart(par):
            pltpu.async_copy(onesb, degT.at[idxs[par]], ssems[par], add=True)

        def swait(par):
            pltpu.make_async_copy(onesb, degT.at[idxs[par]],
                                  ssems[par]).wait()

        def compute(par, sub):
            for i in range(CE // L):
                d16 = dbuf[0, 0, pl.ds(sub * CE + i * L, L)]
                inr = (d16 >= base_c) & (d16 < base_c + HALF)
                idxs[par][pl.ds(i * L, L)] = jnp.where(inr, d16 - base_c,
                                                       DUMP)

        def blk(b, carry):
            blkid = s * NB + b
            pltpu.sync_copy(ep_hbm.at[pl.ds(blkid, 1), pl.ds(1, 1)], dbuf)

            def quad(q, carry2):
                for p in range(NP):
                    @pl.when((b + q) > 0)
                    def _():
                        swait(p)

                    compute(p, q * NP + p)
                    sstart(p)
                return carry2

            lax.fori_loop(0, SCH // NP, quad, 0)
            return carry

        lax.fori_loop(0, NB, blk, 0)
        for p in range(NP):
            swait(p)

        plsc.subcore_barrier()
        rows_pt = HP // NS
        pltpu.sync_copy(degT.at[pl.ds(s * rows_pt, rows_pt)],
                        out_hbm.at[pl.ds(c * HP + s * rows_pt, rows_pt)])

    return k(epack)


def _dense_mid(agg1, degr, h, W1l, b1, W1r, W2l, b2, W2r):
    BR = 128
    GR = PN // BR

    def body(a1_ref, dg_ref, h_ref, w1l_ref, b1_ref, w1r_ref, w2l_ref,
             b2_ref, w2r_ref, p_ref, qb_ref):
        a1 = a1_ref[...]
        deg = jnp.maximum(dg_ref[...][:, :1], 1.0)
        mean = a1 / deg
        h1 = jnp.dot(mean, w1l_ref[...], preferred_element_type=jnp.float32)
        h1 = h1 + b1_ref[...]
        h1 = h1 + jnp.dot(h_ref[...], w1r_ref[...],
                          preferred_element_type=jnp.float32)
        h1 = jnp.maximum(h1, 0.0)
        p_ref[...] = jnp.dot(h1, w2l_ref[...],
                             preferred_element_type=jnp.float32)
        qb_ref[...] = jnp.dot(h1, w2r_ref[...],
                              preferred_element_type=jnp.float32) + b2_ref[...]

    return pl.pallas_call(
        body,
        grid=(GR,),
        in_specs=[
            pl.BlockSpec((BR, D), lambda i: (i, 0)),
            pl.BlockSpec((BR, L), lambda i: (i, 0)),
            pl.BlockSpec((BR, D), lambda i: (i, 0)),
            pl.BlockSpec((D, DH), lambda i: (0, 0)),
            pl.BlockSpec((1, DH), lambda i: (0, 0)),
            pl.BlockSpec((D, DH), lambda i: (0, 0)),
            pl.BlockSpec((DH, D), lambda i: (0, 0)),
            pl.BlockSpec((1, D), lambda i: (0, 0)),
            pl.BlockSpec((DH, D), lambda i: (0, 0)),
        ],
        out_specs=[pl.BlockSpec((BR, D), lambda i: (i, 0)),
                   pl.BlockSpec((BR, D), lambda i: (i, 0))],
        out_shape=[jax.ShapeDtypeStruct((PN, D), jnp.float32),
                   jax.ShapeDtypeStruct((PN, D), jnp.float32)],
    )(agg1, degr, h, W1l, b1.reshape(1, DH), W1r, W2l, b2.reshape(1, D),
      W2r)


def _pool(degr, agg2, qb, batch_pad):
    NSL = SLAB // L
    NPR = NSL // 2

    @functools.partial(
        pl.kernel,
        out_type=jax.ShapeDtypeStruct((NW, OUTR, D), jnp.float32),
        mesh=_mesh(),
        compiler_params=_SC_PARAMS,
        scratch_types=[
            pltpu.VMEM((L, D), jnp.float32),
            pltpu.VMEM((L, D), jnp.float32),
            pltpu.VMEM((L, D), jnp.float32),
            pltpu.VMEM((L, D), jnp.float32),
            pltpu.VMEM((L, L), jnp.float32),
            pltpu.VMEM((L, L), jnp.float32),
            pltpu.VMEM((L,), jnp.int32),
            pltpu.VMEM((L,), jnp.int32),
            pltpu.VMEM((OUTR, D), jnp.float32),
            pltpu.SemaphoreType.DMA,
            pltpu.SemaphoreType.DMA,
        ],
    )
    def k(dg_hbm, a2_hbm, qb_hbm, b_hbm, out_hbm, a2b0, a2b1, qb0, qb1,
          dg0, dg1, bb0, bb1, outl, psem0, psem1):
        c = lax.axis_index("c")
        s = lax.axis_index("s")
        wid = s * NC + c
        base = wid * SLAB
        a2bs = (a2b0, a2b1)
        qbs = (qb0, qb1)
        dgs = (dg0, dg1)
        bbs = (bb0, bb1)
        psems = (psem0, psem1)
        lanes = lax.iota(jnp.int32, L)
        ninf = jnp.full((L,), -jnp.inf, jnp.float32)

        def irow(r, carry):
            for kk in range(D // L):
                outl[r, pl.ds(kk * L, L)] = ninf
            return carry

        lax.fori_loop(0, OUTR, irow, 0)

        def start4(par, slab):
            row0 = base + slab * L
            pltpu.async_copy(a2_hbm.at[pl.ds(row0, L)], a2bs[par],
                             psems[par])
            pltpu.async_copy(qb_hbm.at[pl.ds(row0, L)], qbs[par],
                             psems[par])
            pltpu.async_copy(dg_hbm.at[pl.ds(row0, L)], dgs[par],
                             psems[par])
            pltpu.async_copy(b_hbm.at[pl.ds(row0, L)], bbs[par], psems[par])

        def wait4(par):
            pltpu.make_async_copy(a2_hbm.at[pl.ds(0, L)], a2bs[par],
                                  psems[par]).wait()
            pltpu.make_async_copy(qb_hbm.at[pl.ds(0, L)], qbs[par],
                                  psems[par]).wait()
            pltpu.make_async_copy(dg_hbm.at[pl.ds(0, L)], dgs[par],
                                  psems[par]).wait()
            pltpu.make_async_copy(b_hbm.at[pl.ds(0, L)], bbs[par],
                                  psems[par]).wait()

        def compute(par):
            deg16 = plsc.load_gather(dgs[par],
                                     [lanes, jnp.full((L,), 0, jnp.int32)])
            dinv16 = 1.0 / jnp.maximum(deg16, 1.0)
            b16 = bbs[par][...]
            for j in range(L):
                gj = _bcast(b16, j)
                dj = _bcast(dinv16, j)
                for kk in range(D // L):
                    cols = lanes + kk * L
                    h2v = (a2bs[par][j, pl.ds(kk * L, L)] * dj
                           + qbs[par][j, pl.ds(kk * L, L)])
                    cur = plsc.load_gather(outl, [gj, cols])
                    plsc.store_scatter(outl, [gj, cols],
                                       jnp.maximum(cur, h2v))

        start4(0, 0)

        def pairloop(i, carry):
            wait4(0)
            start4(1, 2 * i + 1)
            compute(0)
            wait4(1)

            @pl.when(i < (NPR - 1))
            def _():
                start4(0, 2 * i + 2)

            compute(1)
            return carry

        lax.fori_loop(0, NPR, pairloop, 0)
        pltpu.sync_copy(outl, out_hbm.at[wid])

    return k(degr, agg2, qb, batch_pad)


def _final_max(parts):
    def body(p_ref, o_ref):
        i = pl.program_id(0)

        @pl.when(i == 0)
        def _():
            o_ref[...] = jnp.full((G, D), -jnp.inf, jnp.float32)

        o_ref[...] = jnp.maximum(o_ref[...], p_ref[0, :G, :])

    return pl.pallas_call(
        body,
        grid=(NW,),
        in_specs=[pl.BlockSpec((1, OUTR, D), lambda i: (i, 0, 0))],
        out_specs=pl.BlockSpec((G, D), lambda i: (0, 0)),
        out_shape=jax.ShapeDtypeStruct((G, D), jnp.float32),
    )(parts)


def kernel(x, edge_index, edge_attr, batch, embed, W1l, b1, W1r, W2l, b2, W2r):
    x = x.astype(jnp.int32)
    src = edge_index[0].astype(jnp.int32)
    dst = edge_index[1].astype(jnp.int32)
    batch = batch.astype(jnp.int32)

    pad_shift = HP - HALF
    x_pad = jnp.zeros((PN,), jnp.int32)
    x_pad = x_pad.at[0:HALF].set(x[:HALF]).at[HP:HP + HALF].set(x[HALF:])
    batch_pad = jnp.full((PN,), G, jnp.int32)
    batch_pad = batch_pad.at[0:HALF].set(batch[:HALF])
    batch_pad = batch_pad.at[HP:HP + HALF].set(batch[HALF:])

    psrc = src + pad_shift * (src >= HALF).astype(jnp.int32)
    npad = EP - E
    psrc = jnp.concatenate([psrc, jnp.zeros((npad,), jnp.int32)])
    dstp = jnp.concatenate([dst, jnp.full((npad,), N, jnp.int32)])
    wp = jnp.concatenate([edge_attr, jnp.zeros((npad,), jnp.float32)])
    epack = jnp.stack(
        [psrc.reshape(TB, CB), dstp.reshape(TB, CB),
         lax.bitcast_convert_type(wp, jnp.int32).reshape(TB, CB)], axis=1)

    h = _embed_gather(embed, x_pad)
    degr = _deg(epack)
    agg1 = _edge_agg(h, epack, degr)
    p, qb = _dense_mid(agg1, degr, h, W1l, b1, W1r, W2l, b2, W2r)
    agg2 = _edge_agg(p, epack, degr)
    parts = _pool(degr, agg2, qb, batch_pad)
    return _final_max(parts)


# speedup vs baseline: 1.4589x; 1.4587x over previous
import functools

import jax
import jax.numpy as jnp
from jax import lax
from jax.experimental import pallas as pl
from jax.experimental.pallas import tpu as pltpu
from jax.experimental.pallas import tpu_sc as plsc

N = 50000
E = 800000
V = 100000
G = 512
D = 64
DH = 128

NC = 2
NS = 16
L = 16
NW = NC * NS

HALF = N // NC
HP = 25088
PN = 2 * HP
DUMP = HALF
SLAB = PN // NW

CE = 128
CB = 2048
EPT = 51200
EP = NS * EPT
NB = EPT // CB
SCH = CB // CE
TB = EP // CB
HBK = CB // 2
TB2 = EP // HBK

OUTR = 520


def _mesh():
    return plsc.VectorSubcoreMesh(core_axis_name="c", subcore_axis_name="s",
                                  num_cores=NC, num_subcores=NS)


_SC_PARAMS = pltpu.CompilerParams(use_tc_tiling_on_sc=False,
                                  needs_layout_passes=False)


def _bcast(vec, j):
    idx = jnp.full((L, 1), j, jnp.int32)
    dnums = lax.GatherDimensionNumbers(
        offset_dims=(), collapsed_slice_dims=(0,), start_index_map=(0,))
    return lax.gather(vec, idx, dnums, (1,),
                      mode=lax.GatherScatterMode.PROMISE_IN_BOUNDS)


def _embed_gather(embed, x_pad):
    GC = 112
    NG = SLAB // GC

    @functools.partial(
        pl.kernel,
        out_type=jax.ShapeDtypeStruct((PN, D), jnp.float32),
        mesh=_mesh(),
        compiler_params=_SC_PARAMS,
        scratch_types=[
            pltpu.VMEM((GC,), jnp.int32),
            pltpu.VMEM((GC, D), jnp.float32),
            pltpu.SemaphoreType.DMA,
        ],
    )
    def k(embed_hbm, x_hbm, h_hbm, idx_v, rows_v, sem):
        c = lax.axis_index("c")
        s = lax.axis_index("s")
        base = (s * NC + c) * SLAB

        def body(g, carry):
            off = base + g * GC
            pltpu.sync_copy(x_hbm.at[pl.ds(off, GC)], idx_v)
            pltpu.async_copy(embed_hbm.at[idx_v], rows_v, sem).wait()
            pltpu.sync_copy(rows_v, h_hbm.at[pl.ds(off, GC)])
            return carry

        lax.fori_loop(0, NG, body, 0)

    return k(embed, x_pad)


def _edge_agg(table, epack, dep):
    EC = 64
    HB = CB // 2

    @functools.partial(
        pl.kernel,
        out_type=jax.ShapeDtypeStruct((PN, D), jnp.float32),
        mesh=_mesh(),
        compiler_params=_SC_PARAMS,
        scratch_types=[
            pltpu.VMEM((1, 3, HB), jnp.int32),
            pltpu.VMEM((CB + EC,), jnp.int32),
            pltpu.VMEM((CB + EC,), jnp.int32),
            pltpu.VMEM((CB + EC,), jnp.int32),
            pltpu.VMEM((EC, D), jnp.float32),
            pltpu.VMEM((EC, D), jnp.float32),
            pltpu.VMEM((EC,), jnp.int32),
            pltpu.VMEM((EC,), jnp.int32),
            pltpu.VMEM_SHARED((HP, D), jnp.float32),
            pltpu.SemaphoreType.DMA,
            pltpu.SemaphoreType.DMA,
            pltpu.SemaphoreType.DMA,
            pltpu.SemaphoreType.DMA,
        ],
    )
    def k(tab_hbm, ep_hbm, dep_hbm, out_hbm, ebuf, src_s, dst_s, w_s,
          rows0, rows1, idx0, idx1, agg, gsem0, gsem1, ssem0, ssem1):
        del dep_hbm
        c = lax.axis_index("c")
        s = lax.axis_index("s")
        base_c = c * HALF
        rows_ = (rows0, rows1)
        idxs = (idx0, idx1)
        gsems = (gsem0, gsem1)
        ssems = (ssem0, ssem1)
        zero = jnp.zeros((L,), jnp.float32)
        dump16 = jnp.full((L,), DUMP, jnp.int32)
        zero_i = jnp.zeros((L,), jnp.int32)

        def zrow(r, carry):
            for kk in range(D // L):
                rows0[r, pl.ds(kk * L, L)] = zero
                rows1[r, pl.ds(kk * L, L)] = zero
            return carry

        lax.fori_loop(0, EC, zrow, 0)

        ZCE = 56

        def zcopy(z, carry):
            pltpu.sync_copy(rows0.at[pl.ds(0, ZCE)],
                            agg.at[pl.ds(s * (HP // NS) + z * ZCE, ZCE)])
            return carry

        lax.fori_loop(0, (HP // NS) // ZCE, zcopy, 0)

        plsc.subcore_barrier()

        def gstart(par, u):
            off = pl.multiple_of(u * EC, EC)
            pltpu.async_copy(tab_hbm.at[src_s.at[pl.ds(off, EC)]],
                             rows_[par], gsems[par])

        def gwait(par):
            pltpu.make_async_copy(tab_hbm.at[idxs[par]], rows_[par],
                                  gsems[par]).wait()

        def sstart(par):
            pltpu.async_copy(rows_[par], agg.at[idxs[par]], ssems[par],
                             add=True)

        def swait(par):
            pltpu.make_async_copy(rows_[par], agg.at[idxs[par]],
                                  ssems[par]).wait()

        for par in range(2):
            for i in range(EC // L):
                idxs[par][pl.ds(i * L, L)] = dump16
            sstart(par)

        def filt(f, cnt):
            d16 = ebuf[0, 1, pl.ds(f * L, L)]
            inr = (d16 >= base_c) & (d16 < base_c + HALF)
            pc = plsc.all_reduce_population_count(inr)
            plsc.store_compressed(src_s.at[pl.ds(cnt, L)],
                                  ebuf[0, 0, pl.ds(f * L, L)], mask=inr)
            plsc.store_compressed(dst_s.at[pl.ds(cnt, L)], d16 - base_c,
                                  mask=inr)
            plsc.store_compressed(w_s.at[pl.ds(cnt, L)],
                                  ebuf[0, 2, pl.ds(f * L, L)], mask=inr)
            return cnt + jnp.max(pc)

        def do_chunk(par, u, nch):
            gwait(par)

            @pl.when(u + 1 < nch)
            def _():
                gstart(1 - par, u + 1)

            swait(par)
            off = pl.multiple_of(u * EC, EC)
            for i in range(EC // L):
                idxs[par][pl.ds(i * L, L)] = dst_s[pl.ds(off + i * L, L)]
                w16 = plsc.bitcast(w_s[pl.ds(off + i * L, L)], jnp.float32)
                for j in range(L):
                    wj = _bcast(w16, j)
                    r = i * L + j
                    for kk in range(D // L):
                        rows_[par][r, pl.ds(kk * L, L)] = (
                            rows_[par][r, pl.ds(kk * L, L)] * wj)
            sstart(par)

        def blk(b, carry):
            blkid = s * NB + b
            cnt = jnp.int32(0)
            pltpu.sync_copy(ep_hbm.at[pl.ds(blkid * 2, 1)], ebuf)
            cnt = lax.fori_loop(0, HB // L, filt, cnt)
            pltpu.sync_copy(ep_hbm.at[pl.ds(blkid * 2 + 1, 1)], ebuf)
            cnt = lax.fori_loop(0, HB // L, filt, cnt)

            for t in range(EC // L):
                pos = pl.ds(cnt + t * L, L)
                src_s[pos] = zero_i
                dst_s[pos] = dump16
                w_s[pos] = zero_i

            nch = (cnt + (EC - 1)) // EC

            @pl.when(nch > 0)
            def _():
                gstart(0, 0)

            def chunk_loop(u, carry2):
                @pl.when(u % 2 == 0)
                def _():
                    do_chunk(0, u, nch)

                @pl.when(u % 2 == 1)
                def _():
                    do_chunk(1, u, nch)

                return carry2

            lax.fori_loop(0, nch, chunk_loop, 0)
            return carry

        lax.fori_loop(0, NB, blk, 0)
        swait(0)
        swait(1)

        plsc.subcore_barrier()
        rows_pt = HP // NS
        pltpu.sync_copy(agg.at[pl.ds(s * rows_pt, rows_pt)],
                        out_hbm.at[pl.ds(c * HP + s * rows_pt, rows_pt)])

    return k(table, epack, dep)


def _deg(epack):
    ZC = 112
    NZ = (HP // NS) // ZC
    NP = 4

    @functools.partial(
        pl.kernel,
        out_type=jax.ShapeDtypeStruct((PN, L), jnp.float32),
        mesh=_mesh(),
        compiler_params=_SC_PARAMS,
        scratch_types=[
            pltpu.VMEM((1, 1, HBK), jnp.int32),
            pltpu.VMEM((CE, L), jnp.float32),
            pltpu.VMEM((CE,), jnp.int32),
            pltpu.VMEM((CE,), jnp.int32),
            pltpu.VMEM((CE,), jnp.int32),
            pltpu.VMEM((CE,), jnp.int32),
            pltpu.VMEM_SHARED((HP, L), jnp.float32),
            pltpu.SemaphoreType.DMA,
            pltpu.SemaphoreType.DMA,
            pltpu.SemaphoreType.DMA,
            pltpu.SemaphoreType.DMA,
        ],
    )
    def k(ep_hbm, out_hbm, dbuf, onesb, idx0, idx1, idx2, idx3, degT,
          ssem0, ssem1, ssem2, ssem3):
        c = lax.axis_index("c")
        s = lax.axis_index("s")
        base_c = c * HALF
        idxs = (idx0, idx1, idx2, idx3)
        ssems = (ssem0, ssem1, ssem2, ssem3)
        zero = jnp.zeros((L,), jnp.float32)

        def zrow(r, carry):
            onesb[r, pl.ds(0, L)] = zero
            return carry

        lax.fori_loop(0, CE, zrow, 0)

        def zcopy(z, carry):
            pltpu.sync_copy(onesb.at[pl.ds(0, ZC)],
                            degT.at[pl.ds(s * (HP // NS) + z * ZC, ZC)])
            return carry

        lax.fori_loop(0, NZ, zcopy, 0)

        ones = jnp.ones((L,), jnp.float32)

        def orow(r, carry):
            onesb[r, pl.ds(0, L)] = ones
            return carry

        lax.fori_loop(0, CE, orow, 0)

        plsc.subcore_barrier()

        def sstart(par):
            pltpu.async_copy(onesb, degT.at[idxs[par]], ssems[par], add=True)

        def swait(par):
            pltpu.make_async_copy(onesb, degT.at[idxs[par]],
                                  ssems[par]).wait()

        def compute(par, sub):
            for i in range(CE // L):
                d16 = dbuf[0, 0, pl.ds(sub * CE + i * L, L)]
                inr = (d16 >= base_c) & (d16 < base_c + HALF)
                idxs[par][pl.ds(i * L, L)] = jnp.where(inr, d16 - base_c,
                                                       DUMP)

        NB2 = EPT // HBK
        SCH2 = HBK // CE

        def blk(b, carry):
            blkid = s * NB2 + b
            pltpu.sync_copy(ep_hbm.at[pl.ds(blkid, 1), pl.ds(1, 1)], dbuf)

            def quad(q, carry2):
                for p in range(NP):
                    @pl.when((b + q) > 0)
                    def _():
                        swait(p)

                    compute(p, q * NP + p)
                    sstart(p)
                return carry2

            lax.fori_loop(0, SCH2 // NP, quad, 0)
            return carry

        lax.fori_loop(0, NB2, blk, 0)
        for p in range(NP):
            swait(p)

        plsc.subcore_barrier()
        rows_pt = HP // NS
        pltpu.sync_copy(degT.at[pl.ds(s * rows_pt, rows_pt)],
                        out_hbm.at[pl.ds(c * HP + s * rows_pt, rows_pt)])

    return k(epack)


def _dense_mid(agg1, degr, h, W1l, b1, W1r, W2l, b2, W2r):
    BR = 128
    GR = PN // BR

    def body(a1_ref, dg_ref, h_ref, w1l_ref, b1_ref, w1r_ref, w2l_ref,
             b2_ref, w2r_ref, p_ref, qb_ref):
        a1 = a1_ref[...]
        deg = jnp.maximum(dg_ref[...][:, :1], 1.0)
        mean = a1 / deg
        h1 = jnp.dot(mean, w1l_ref[...], preferred_element_type=jnp.float32)
        h1 = h1 + b1_ref[...]
        h1 = h1 + jnp.dot(h_ref[...], w1r_ref[...],
                          preferred_element_type=jnp.float32)
        h1 = jnp.maximum(h1, 0.0)
        p_ref[...] = jnp.dot(h1, w2l_ref[...],
                             preferred_element_type=jnp.float32)
        qb_ref[...] = jnp.dot(h1, w2r_ref[...],
                              preferred_element_type=jnp.float32) + b2_ref[...]

    return pl.pallas_call(
        body,
        grid=(GR,),
        in_specs=[
            pl.BlockSpec((BR, D), lambda i: (i, 0)),
            pl.BlockSpec((BR, L), lambda i: (i, 0)),
            pl.BlockSpec((BR, D), lambda i: (i, 0)),
            pl.BlockSpec((D, DH), lambda i: (0, 0)),
            pl.BlockSpec((1, DH), lambda i: (0, 0)),
            pl.BlockSpec((D, DH), lambda i: (0, 0)),
            pl.BlockSpec((DH, D), lambda i: (0, 0)),
            pl.BlockSpec((1, D), lambda i: (0, 0)),
            pl.BlockSpec((DH, D), lambda i: (0, 0)),
        ],
        out_specs=[pl.BlockSpec((BR, D), lambda i: (i, 0)),
                   pl.BlockSpec((BR, D), lambda i: (i, 0))],
        out_shape=[jax.ShapeDtypeStruct((PN, D), jnp.float32),
                   jax.ShapeDtypeStruct((PN, D), jnp.float32)],
    )(agg1, degr, h, W1l, b1.reshape(1, DH), W1r, W2l, b2.reshape(1, D),
      W2r)


def _pool(degr, agg2, qb, batch_pad):
    NSL = SLAB // L
    NPR = NSL // 2

    @functools.partial(
        pl.kernel,
        out_type=jax.ShapeDtypeStruct((NW, OUTR, D), jnp.float32),
        mesh=_mesh(),
        compiler_params=_SC_PARAMS,
        scratch_types=[
            pltpu.VMEM((L, D), jnp.float32),
            pltpu.VMEM((L, D), jnp.float32),
            pltpu.VMEM((L, D), jnp.float32),
            pltpu.VMEM((L, D), jnp.float32),
            pltpu.VMEM((L, L), jnp.float32),
            pltpu.VMEM((L, L), jnp.float32),
            pltpu.VMEM((L,), jnp.int32),
            pltpu.VMEM((L,), jnp.int32),
            pltpu.VMEM((OUTR, D), jnp.float32),
            pltpu.SemaphoreType.DMA,
            pltpu.SemaphoreType.DMA,
        ],
    )
    def k(dg_hbm, a2_hbm, qb_hbm, b_hbm, out_hbm, a2b0, a2b1, qb0, qb1,
          dg0, dg1, bb0, bb1, outl, psem0, psem1):
        c = lax.axis_index("c")
        s = lax.axis_index("s")
        wid = s * NC + c
        base = wid * SLAB
        a2bs = (a2b0, a2b1)
        qbs = (qb0, qb1)
        dgs = (dg0, dg1)
        bbs = (bb0, bb1)
        psems = (psem0, psem1)
        lanes = lax.iota(jnp.int32, L)
        ninf = jnp.full((L,), -jnp.inf, jnp.float32)

        def irow(r, carry):
            for kk in range(D // L):
                outl[r, pl.ds(kk * L, L)] = ninf
            return carry

        lax.fori_loop(0, OUTR, irow, 0)

        def start4(par, slab):
            row0 = base + slab * L
            pltpu.async_copy(a2_hbm.at[pl.ds(row0, L)], a2bs[par],
                             psems[par])
            pltpu.async_copy(qb_hbm.at[pl.ds(row0, L)], qbs[par],
                             psems[par])
            pltpu.async_copy(dg_hbm.at[pl.ds(row0, L)], dgs[par],
                             psems[par])
            pltpu.async_copy(b_hbm.at[pl.ds(row0, L)], bbs[par], psems[par])

        def wait4(par):
            pltpu.make_async_copy(a2_hbm.at[pl.ds(0, L)], a2bs[par],
                                  psems[par]).wait()
            pltpu.make_async_copy(qb_hbm.at[pl.ds(0, L)], qbs[par],
                                  psems[par]).wait()
            pltpu.make_async_copy(dg_hbm.at[pl.ds(0, L)], dgs[par],
                                  psems[par]).wait()
            pltpu.make_async_copy(b_hbm.at[pl.ds(0, L)], bbs[par],
                                  psems[par]).wait()

        def compute(par):
            deg16 = plsc.load_gather(dgs[par],
                                     [lanes, jnp.full((L,), 0, jnp.int32)])
            dinv16 = 1.0 / jnp.maximum(deg16, 1.0)
            b16 = bbs[par][...]
            for j in range(L):
                gj = _bcast(b16, j)
                dj = _bcast(dinv16, j)
                for kk in range(D // L):
                    cols = lanes + kk * L
                    h2v = (a2bs[par][j, pl.ds(kk * L, L)] * dj
                           + qbs[par][j, pl.ds(kk * L, L)])
                    cur = plsc.load_gather(outl, [gj, cols])
                    plsc.store_scatter(outl, [gj, cols],
                                       jnp.maximum(cur, h2v))

        start4(0, 0)

        def pairloop(i, carry):
            wait4(0)
            start4(1, 2 * i + 1)
            compute(0)
            wait4(1)

            @pl.when(i < (NPR - 1))
            def _():
                start4(0, 2 * i + 2)

            compute(1)
            return carry

        lax.fori_loop(0, NPR, pairloop, 0)
        pltpu.sync_copy(outl, out_hbm.at[wid])

    return k(degr, agg2, qb, batch_pad)


def _final_max(parts):
    def body(p_ref, o_ref):
        i = pl.program_id(0)

        @pl.when(i == 0)
        def _():
            o_ref[...] = jnp.full((G, D), -jnp.inf, jnp.float32)

        o_ref[...] = jnp.maximum(o_ref[...], p_ref[0, :G, :])

    return pl.pallas_call(
        body,
        grid=(NW,),
        in_specs=[pl.BlockSpec((1, OUTR, D), lambda i: (i, 0, 0))],
        out_specs=pl.BlockSpec((G, D), lambda i: (0, 0)),
        out_shape=jax.ShapeDtypeStruct((G, D), jnp.float32),
    )(parts)


def kernel(x, edge_index, edge_attr, batch, embed, W1l, b1, W1r, W2l, b2, W2r):
    x = x.astype(jnp.int32)
    src = edge_index[0].astype(jnp.int32)
    dst = edge_index[1].astype(jnp.int32)
    batch = batch.astype(jnp.int32)

    pad_shift = HP - HALF
    x_pad = jnp.zeros((PN,), jnp.int32)
    x_pad = x_pad.at[0:HALF].set(x[:HALF]).at[HP:HP + HALF].set(x[HALF:])
    batch_pad = jnp.full((PN,), G, jnp.int32)
    batch_pad = batch_pad.at[0:HALF].set(batch[:HALF])
    batch_pad = batch_pad.at[HP:HP + HALF].set(batch[HALF:])

    psrc = src + pad_shift * (src >= HALF).astype(jnp.int32)
    npad = EP - E
    psrc = jnp.concatenate([psrc, jnp.zeros((npad,), jnp.int32)])
    dstp = jnp.concatenate([dst, jnp.full((npad,), N, jnp.int32)])
    wp = jnp.concatenate([edge_attr, jnp.zeros((npad,), jnp.float32)])
    epack = jnp.stack(
        [psrc.reshape(TB2, HBK), dstp.reshape(TB2, HBK),
         lax.bitcast_convert_type(wp, jnp.int32).reshape(TB2, HBK)], axis=1)

    h = _embed_gather(embed, x_pad)
    degr = _deg(epack)
    agg1 = _edge_agg(h, epack, degr)
    p, qb = _dense_mid(agg1, degr, h, W1l, b1, W1r, W2l, b2, W2r)
    agg2 = _edge_agg(p, epack, degr)
    parts = _pool(degr, agg2, qb, batch_pad)
    return _final_max(parts)


# speedup vs baseline: 1.7545x; 1.2026x over previous
import functools

import jax
import jax.numpy as jnp
from jax import lax
from jax.experimental import pallas as pl
from jax.experimental.pallas import tpu as pltpu
from jax.experimental.pallas import tpu_sc as plsc

N = 50000
E = 800000
V = 100000
G = 512
D = 64
DH = 128

NC = 2
NS = 16
L = 16
NW = NC * NS

HALF = N // NC
HP = 25088
PN = 2 * HP
DUMP = HALF
SLAB = PN // NW

CE = 128
CB = 2048
EPT = 51200
EP = NS * EPT
NB = EPT // CB
SCH = CB // CE
TB = EP // CB
HBK = CB // 2
TB2 = EP // HBK

OUTR = 520


def _mesh():
    return plsc.VectorSubcoreMesh(core_axis_name="c", subcore_axis_name="s",
                                  num_cores=NC, num_subcores=NS)


_SC_PARAMS = pltpu.CompilerParams(use_tc_tiling_on_sc=False,
                                  needs_layout_passes=False)


def _bcast(vec, j):
    idx = jnp.full((L, 1), j, jnp.int32)
    dnums = lax.GatherDimensionNumbers(
        offset_dims=(), collapsed_slice_dims=(0,), start_index_map=(0,))
    return lax.gather(vec, idx, dnums, (1,),
                      mode=lax.GatherScatterMode.PROMISE_IN_BOUNDS)


def _embed_gather(embed, x_pad):
    GC = 112
    NG = SLAB // GC

    @functools.partial(
        pl.kernel,
        out_type=jax.ShapeDtypeStruct((PN, D), jnp.float32),
        mesh=_mesh(),
        compiler_params=_SC_PARAMS,
        scratch_types=[
            pltpu.VMEM((GC,), jnp.int32),
            pltpu.VMEM((GC, D), jnp.float32),
            pltpu.SemaphoreType.DMA,
        ],
    )
    def k(embed_hbm, x_hbm, h_hbm, idx_v, rows_v, sem):
        c = lax.axis_index("c")
        s = lax.axis_index("s")
        base = (s * NC + c) * SLAB

        def body(g, carry):
            off = base + g * GC
            pltpu.sync_copy(x_hbm.at[pl.ds(off, GC)], idx_v)
            pltpu.async_copy(embed_hbm.at[idx_v], rows_v, sem).wait()
            pltpu.sync_copy(rows_v, h_hbm.at[pl.ds(off, GC)])
            return carry

        lax.fori_loop(0, NG, body, 0)

    return k(embed, x_pad)


def _edge_agg(table, epack, dep):
    EC = 64
    HB = CB // 2

    @functools.partial(
        pl.kernel,
        out_type=jax.ShapeDtypeStruct((PN, D), jnp.float32),
        mesh=_mesh(),
        compiler_params=_SC_PARAMS,
        scratch_types=[
            pltpu.VMEM((1, 3, HB), jnp.int32),
            pltpu.VMEM((CB + EC,), jnp.int32),
            pltpu.VMEM((CB + EC,), jnp.int32),
            pltpu.VMEM((CB + EC,), jnp.int32),
            pltpu.VMEM((EC, D), jnp.float32),
            pltpu.VMEM((EC, D), jnp.float32),
            pltpu.VMEM((EC,), jnp.int32),
            pltpu.VMEM((EC,), jnp.int32),
            pltpu.VMEM_SHARED((HP, D), jnp.float32),
            pltpu.SemaphoreType.DMA,
            pltpu.SemaphoreType.DMA,
            pltpu.SemaphoreType.DMA,
            pltpu.SemaphoreType.DMA,
        ],
    )
    def k(tab_hbm, ep_hbm, dep_hbm, out_hbm, ebuf, src_s, dst_s, w_s,
          rows0, rows1, idx0, idx1, agg, gsem0, gsem1, ssem0, ssem1):
        del dep_hbm
        c = lax.axis_index("c")
        s = lax.axis_index("s")
        base_c = c * HALF
        rows_ = (rows0, rows1)
        idxs = (idx0, idx1)
        gsems = (gsem0, gsem1)
        ssems = (ssem0, ssem1)
        zero = jnp.zeros((L,), jnp.float32)
        dump16 = jnp.full((L,), DUMP, jnp.int32)
        zero_i = jnp.zeros((L,), jnp.int32)

        def zrow(r, carry):
            for kk in range(D // L):
                rows0[r, pl.ds(kk * L, L)] = zero
                rows1[r, pl.ds(kk * L, L)] = zero
            return carry

        lax.fori_loop(0, EC, zrow, 0)

        ZCE = 56

        def zcopy(z, carry):
            pltpu.sync_copy(rows0.at[pl.ds(0, ZCE)],
                            agg.at[pl.ds(s * (HP // NS) + z * ZCE, ZCE)])
            return carry

        lax.fori_loop(0, (HP // NS) // ZCE, zcopy, 0)

        plsc.subcore_barrier()

        def gstart(par, u):
            off = pl.multiple_of(u * EC, EC)
            pltpu.async_copy(tab_hbm.at[src_s.at[pl.ds(off, EC)]],
                             rows_[par], gsems[par])

        def gwait(par):
            pltpu.make_async_copy(tab_hbm.at[idxs[par]], rows_[par],
                                  gsems[par]).wait()

        def sstart(par):
            pltpu.async_copy(rows_[par], agg.at[idxs[par]], ssems[par],
                             add=True)

        def swait(par):
            pltpu.make_async_copy(rows_[par], agg.at[idxs[par]],
                                  ssems[par]).wait()

        for par in range(2):
            for i in range(EC // L):
                idxs[par][pl.ds(i * L, L)] = dump16
            sstart(par)

        def filt(f, cnt):
            d16 = ebuf[0, 1, pl.ds(f * L, L)]
            inr = (d16 >= base_c) & (d16 < base_c + HALF)
            pc = plsc.all_reduce_population_count(inr)
            plsc.store_compressed(src_s.at[pl.ds(cnt, L)],
                                  ebuf[0, 0, pl.ds(f * L, L)], mask=inr)
            plsc.store_compressed(dst_s.at[pl.ds(cnt, L)], d16 - base_c,
                                  mask=inr)
            plsc.store_compressed(w_s.at[pl.ds(cnt, L)],
                                  ebuf[0, 2, pl.ds(f * L, L)], mask=inr)
            return cnt + jnp.max(pc)

        def do_chunk(par, u, nch):
            gwait(par)

            @pl.when(u + 1 < nch)
            def _():
                gstart(1 - par, u + 1)

            swait(par)
            off = pl.multiple_of(u * EC, EC)
            for i in range(EC // L):
                idxs[par][pl.ds(i * L, L)] = dst_s[pl.ds(off + i * L, L)]
                w16 = plsc.bitcast(w_s[pl.ds(off + i * L, L)], jnp.float32)
                for j in range(L):
                    wj = _bcast(w16, j)
                    r = i * L + j
                    for kk in range(D // L):
                        rows_[par][r, pl.ds(kk * L, L)] = (
                            rows_[par][r, pl.ds(kk * L, L)] * wj)
            sstart(par)

        def blk(b, carry):
            blkid = s * NB + b
            cnt = jnp.int32(0)
            pltpu.sync_copy(ep_hbm.at[pl.ds(blkid * 2, 1)], ebuf)
            cnt = lax.fori_loop(0, HB // L, filt, cnt)
            pltpu.sync_copy(ep_hbm.at[pl.ds(blkid * 2 + 1, 1)], ebuf)
            cnt = lax.fori_loop(0, HB // L, filt, cnt)

            for t in range(EC // L):
                pos = pl.ds(cnt + t * L, L)
                src_s[pos] = zero_i
                dst_s[pos] = dump16
                w_s[pos] = zero_i

            nch = (cnt + (EC - 1)) // EC

            @pl.when(nch > 0)
            def _():
                gstart(0, 0)

            def chunk_loop(u, carry2):
                @pl.when(u % 2 == 0)
                def _():
                    do_chunk(0, u, nch)

                @pl.when(u % 2 == 1)
                def _():
                    do_chunk(1, u, nch)

                return carry2

            lax.fori_loop(0, nch, chunk_loop, 0)
            return carry

        lax.fori_loop(0, NB, blk, 0)
        swait(0)
        swait(1)

        plsc.subcore_barrier()
        rows_pt = HP // NS
        pltpu.sync_copy(agg.at[pl.ds(s * rows_pt, rows_pt)],
                        out_hbm.at[pl.ds(c * HP + s * rows_pt, rows_pt)])

    return k(table, epack, dep)


def _deg(epack):
    ZC = 112
    NZ = (HP // NS) // ZC
    NB2 = EPT // HBK
    HB = HBK

    @functools.partial(
        pl.kernel,
        out_type=jax.ShapeDtypeStruct((PN, L), jnp.float32),
        mesh=_mesh(),
        compiler_params=_SC_PARAMS,
        scratch_types=[
            pltpu.VMEM((1, 1, HBK), jnp.int32),
            pltpu.VMEM((HBK + CE,), jnp.int32),
            pltpu.VMEM((CE, L), jnp.float32),
            pltpu.VMEM((CE,), jnp.int32),
            pltpu.VMEM((CE,), jnp.int32),
            pltpu.VMEM_SHARED((HP, L), jnp.float32),
            pltpu.SemaphoreType.DMA,
            pltpu.SemaphoreType.DMA,
        ],
    )
    def k(ep_hbm, out_hbm, dbuf, dst_s, onesb, idx0, idx1, degT,
          ssem0, ssem1):
        c = lax.axis_index("c")
        s = lax.axis_index("s")
        base_c = c * HALF
        idxs = (idx0, idx1)
        ssems = (ssem0, ssem1)
        zero = jnp.zeros((L,), jnp.float32)
        dump16 = jnp.full((L,), DUMP, jnp.int32)

        def zrow(r, carry):
            onesb[r, pl.ds(0, L)] = zero
            return carry

        lax.fori_loop(0, CE, zrow, 0)

        def zcopy(z, carry):
            pltpu.sync_copy(onesb.at[pl.ds(0, ZC)],
                            degT.at[pl.ds(s * (HP // NS) + z * ZC, ZC)])
            return carry

        lax.fori_loop(0, NZ, zcopy, 0)

        ones = jnp.ones((L,), jnp.float32)

        def orow(r, carry):
            onesb[r, pl.ds(0, L)] = ones
            return carry

        lax.fori_loop(0, CE, orow, 0)

        plsc.subcore_barrier()

        def sstart(par):
            pltpu.async_copy(onesb, degT.at[idxs[par]], ssems[par], add=True)

        def swait(par):
            pltpu.make_async_copy(onesb, degT.at[idxs[par]],
                                  ssems[par]).wait()

        for par in range(2):
            for i in range(CE // L):
                idxs[par][pl.ds(i * L, L)] = dump16
            sstart(par)

        def filt(f, cnt):
            d16 = dbuf[0, 0, pl.ds(f * L, L)]
            inr = (d16 >= base_c) & (d16 < base_c + HALF)
            pc = plsc.all_reduce_population_count(inr)
            plsc.store_compressed(dst_s.at[pl.ds(cnt, L)], d16 - base_c,
                                  mask=inr)
            return cnt + jnp.max(pc)

        def do_chunk(par, u):
            swait(par)
            off = pl.multiple_of(u * CE, CE)
            for i in range(CE // L):
                idxs[par][pl.ds(i * L, L)] = dst_s[pl.ds(off + i * L, L)]
            sstart(par)

        def blk(b, carry):
            blkid = s * NB2 + b
            cnt = jnp.int32(0)
            pltpu.sync_copy(ep_hbm.at[pl.ds(blkid, 1), pl.ds(1, 1)], dbuf)
            cnt = lax.fori_loop(0, HB // L, filt, cnt)

            for t in range(CE // L):
                dst_s[pl.ds(cnt + t * L, L)] = dump16

            nch = (cnt + (CE - 1)) // CE

            def chunk_loop(u, carry2):
                @pl.when(u % 2 == 0)
                def _():
                    do_chunk(0, u)

                @pl.when(u % 2 == 1)
                def _():
                    do_chunk(1, u)

                return carry2

            lax.fori_loop(0, nch, chunk_loop, 0)
            return carry

        lax.fori_loop(0, NB2, blk, 0)
        swait(0)
        swait(1)

        plsc.subcore_barrier()
        rows_pt = HP // NS
        pltpu.sync_copy(degT.at[pl.ds(s * rows_pt, rows_pt)],
                        out_hbm.at[pl.ds(c * HP + s * rows_pt, rows_pt)])

    return k(epack)


def _dense_mid(agg1, degr, h, W1l, b1, W1r, W2l, b2, W2r):
    BR = 128
    GR = PN // BR

    def body(a1_ref, dg_ref, h_ref, w1l_ref, b1_ref, w1r_ref, w2l_ref,
             b2_ref, w2r_ref, p_ref, qb_ref):
        a1 = a1_ref[...]
        deg = jnp.maximum(dg_ref[...][:, :1], 1.0)
        mean = a1 / deg
        h1 = jnp.dot(mean, w1l_ref[...], preferred_element_type=jnp.float32)
        h1 = h1 + b1_ref[...]
        h1 = h1 + jnp.dot(h_ref[...], w1r_ref[...],
                          preferred_element_type=jnp.float32)
        h1 = jnp.maximum(h1, 0.0)
        p_ref[...] = jnp.dot(h1, w2l_ref[...],
                             preferred_element_type=jnp.float32)
        qb_ref[...] = jnp.dot(h1, w2r_ref[...],
                              preferred_element_type=jnp.float32) + b2_ref[...]

    return pl.pallas_call(
        body,
        grid=(GR,),
        in_specs=[
            pl.BlockSpec((BR, D), lambda i: (i, 0)),
            pl.BlockSpec((BR, L), lambda i: (i, 0)),
            pl.BlockSpec((BR, D), lambda i: (i, 0)),
            pl.BlockSpec((D, DH), lambda i: (0, 0)),
            pl.BlockSpec((1, DH), lambda i: (0, 0)),
            pl.BlockSpec((D, DH), lambda i: (0, 0)),
            pl.BlockSpec((DH, D), lambda i: (0, 0)),
            pl.BlockSpec((1, D), lambda i: (0, 0)),
            pl.BlockSpec((DH, D), lambda i: (0, 0)),
        ],
        out_specs=[pl.BlockSpec((BR, D), lambda i: (i, 0)),
                   pl.BlockSpec((BR, D), lambda i: (i, 0))],
        out_shape=[jax.ShapeDtypeStruct((PN, D), jnp.float32),
                   jax.ShapeDtypeStruct((PN, D), jnp.float32)],
    )(agg1, degr, h, W1l, b1.reshape(1, DH), W1r, W2l, b2.reshape(1, D),
      W2r)


def _pool(degr, agg2, qb, batch_pad):
    NSL = SLAB // L
    NPR = NSL // 2

    @functools.partial(
        pl.kernel,
        out_type=jax.ShapeDtypeStruct((NW, OUTR, D), jnp.float32),
        mesh=_mesh(),
        compiler_params=_SC_PARAMS,
        scratch_types=[
            pltpu.VMEM((L, D), jnp.float32),
            pltpu.VMEM((L, D), jnp.float32),
            pltpu.VMEM((L, D), jnp.float32),
            pltpu.VMEM((L, D), jnp.float32),
            pltpu.VMEM((L, L), jnp.float32),
            pltpu.VMEM((L, L), jnp.float32),
            pltpu.VMEM((L,), jnp.int32),
            pltpu.VMEM((L,), jnp.int32),
            pltpu.VMEM((OUTR, D), jnp.float32),
            pltpu.SemaphoreType.DMA,
            pltpu.SemaphoreType.DMA,
        ],
    )
    def k(dg_hbm, a2_hbm, qb_hbm, b_hbm, out_hbm, a2b0, a2b1, qb0, qb1,
          dg0, dg1, bb0, bb1, outl, psem0, psem1):
        c = lax.axis_index("c")
        s = lax.axis_index("s")
        wid = s * NC + c
        base = wid * SLAB
        a2bs = (a2b0, a2b1)
        qbs = (qb0, qb1)
        dgs = (dg0, dg1)
        bbs = (bb0, bb1)
        psems = (psem0, psem1)
        lanes = lax.iota(jnp.int32, L)
        ninf = jnp.full((L,), -jnp.inf, jnp.float32)

        def irow(r, carry):
            for kk in range(D // L):
                outl[r, pl.ds(kk * L, L)] = ninf
            return carry

        lax.fori_loop(0, OUTR, irow, 0)

        def start4(par, slab):
            row0 = base + slab * L
            pltpu.async_copy(a2_hbm.at[pl.ds(row0, L)], a2bs[par],
                             psems[par])
            pltpu.async_copy(qb_hbm.at[pl.ds(row0, L)], qbs[par],
                             psems[par])
            pltpu.async_copy(dg_hbm.at[pl.ds(row0, L)], dgs[par],
                             psems[par])
            pltpu.async_copy(b_hbm.at[pl.ds(row0, L)], bbs[par], psems[par])

        def wait4(par):
            pltpu.make_async_copy(a2_hbm.at[pl.ds(0, L)], a2bs[par],
                                  psems[par]).wait()
            pltpu.make_async_copy(qb_hbm.at[pl.ds(0, L)], qbs[par],
                                  psems[par]).wait()
            pltpu.make_async_copy(dg_hbm.at[pl.ds(0, L)], dgs[par],
                                  psems[par]).wait()
            pltpu.make_async_copy(b_hbm.at[pl.ds(0, L)], bbs[par],
                                  psems[par]).wait()

        def compute(par):
            deg16 = plsc.load_gather(dgs[par],
                                     [lanes, jnp.full((L,), 0, jnp.int32)])
            dinv16 = 1.0 / jnp.maximum(deg16, 1.0)
            b16 = bbs[par][...]
            for j in range(L):
                gj = _bcast(b16, j)
                dj = _bcast(dinv16, j)
                for kk in range(D // L):
                    cols = lanes + kk * L
                    h2v = (a2bs[par][j, pl.ds(kk * L, L)] * dj
                           + qbs[par][j, pl.ds(kk * L, L)])
                    cur = plsc.load_gather(outl, [gj, cols])
                    plsc.store_scatter(outl, [gj, cols],
                                       jnp.maximum(cur, h2v))

        start4(0, 0)

        def pairloop(i, carry):
            wait4(0)
            start4(1, 2 * i + 1)
            compute(0)
            wait4(1)

            @pl.when(i < (NPR - 1))
            def _():
                start4(0, 2 * i + 2)

            compute(1)
            return carry

        lax.fori_loop(0, NPR, pairloop, 0)
        pltpu.sync_copy(outl, out_hbm.at[wid])

    return k(degr, agg2, qb, batch_pad)


def _final_max(parts):
    def body(p_ref, o_ref):
        i = pl.program_id(0)

        @pl.when(i == 0)
        def _():
            o_ref[...] = jnp.full((G, D), -jnp.inf, jnp.float32)

        o_ref[...] = jnp.maximum(o_ref[...], p_ref[0, :G, :])

    return pl.pallas_call(
        body,
        grid=(NW,),
        in_specs=[pl.BlockSpec((1, OUTR, D), lambda i: (i, 0, 0))],
        out_specs=pl.BlockSpec((G, D), lambda i: (0, 0)),
        out_shape=jax.ShapeDtypeStruct((G, D), jnp.float32),
    )(parts)


def kernel(x, edge_index, edge_attr, batch, embed, W1l, b1, W1r, W2l, b2, W2r):
    x = x.astype(jnp.int32)
    src = edge_index[0].astype(jnp.int32)
    dst = edge_index[1].astype(jnp.int32)
    batch = batch.astype(jnp.int32)

    pad_shift = HP - HALF
    x_pad = jnp.zeros((PN,), jnp.int32)
    x_pad = x_pad.at[0:HALF].set(x[:HALF]).at[HP:HP + HALF].set(x[HALF:])
    batch_pad = jnp.full((PN,), G, jnp.int32)
    batch_pad = batch_pad.at[0:HALF].set(batch[:HALF])
    batch_pad = batch_pad.at[HP:HP + HALF].set(batch[HALF:])

    psrc = src + pad_shift * (src >= HALF).astype(jnp.int32)
    npad = EP - E
    psrc = jnp.concatenate([psrc, jnp.zeros((npad,), jnp.int32)])
    dstp = jnp.concatenate([dst, jnp.full((npad,), N, jnp.int32)])
    wp = jnp.concatenate([edge_attr, jnp.zeros((npad,), jnp.float32)])
    epack = jnp.stack(
        [psrc.reshape(TB2, HBK), dstp.reshape(TB2, HBK),
         lax.bitcast_convert_type(wp, jnp.int32).reshape(TB2, HBK)], axis=1)

    h = _embed_gather(embed, x_pad)
    degr = _deg(epack)
    agg1 = _edge_agg(h, epack, degr)
    p, qb = _dense_mid(agg1, degr, h, W1l, b1, W1r, W2l, b2, W2r)
    agg2 = _edge_agg(p, epack, degr)
    parts = _pool(degr, agg2, qb, batch_pad)
    return _final_max(parts)
